# baseline (speedup 1.0000x reference)
import numpy as np
import jax
import jax.numpy as jnp
from jax import lax
from jax.experimental import pallas as pl
from jax.experimental.pallas import tpu as pltpu
from jax.experimental.pallas import tpu_sc as plsc

_VOCAB = 100000
_D = 768
_SEQ = 2048
_BATCH = 4

_NC, _NS = 2, 16
_NW = _NC * _NS
_K = 2
_SP = _SEQ // _K
_PPW = _SP // _NW
_CHUNK = _PPW


def _pe_table() -> np.ndarray:
    position = np.arange(0.0, _SEQ)[:, None]
    div_term = np.exp(np.arange(0.0, _D, 2) * -(np.log(10000.0) / _D))
    pe = np.zeros((_SEQ, _D), dtype=np.float32)
    pe[:, 0::2] = np.sin(position * div_term)
    pe[:, 1::2] = np.cos(position * div_term)
    return pe[:, None, :]


_PE = _pe_table()

_mesh = plsc.VectorSubcoreMesh(core_axis_name="c", subcore_axis_name="s")


def _gather(idx_flat, table):
    @pl.kernel(
        out_type=jax.ShapeDtypeStruct((_BATCH, _SP, _D), jnp.float32),
        mesh=_mesh,
        scratch_types=[
            pltpu.VMEM((_BATCH * _PPW,), jnp.int32),
            pltpu.VMEM((_CHUNK, _D), jnp.float32),
            pltpu.VMEM((_CHUNK, _D), jnp.float32),
            pltpu.SemaphoreType.DMA((_BATCH,)),
            pltpu.SemaphoreType.DMA((2,)),
            pltpu.SemaphoreType.DMA((2,)),
        ],
    )
    def body(idx_hbm, table_hbm, out_hbm, idx_v, buf_v0, buf_v1,
             sem_i, sem_g, sem_w):
        buf_v = [buf_v0, buf_v1]
        wid = lax.axis_index("s") * _NC + lax.axis_index("c")
        pbase = wid * _PPW
        idx_cp = [
            pltpu.async_copy(
                idx_hbm.at[pl.ds(b * _SP + pbase, _PPW)],
                idx_v.at[pl.ds(b * _PPW, _PPW)], sem_i.at[b])
            for b in range(_BATCH)
        ]

        def start_gather(b):
            idx_cp[b].wait()
            return pltpu.async_copy(
                table_hbm.at[idx_v.at[pl.ds(b * _PPW, _CHUNK)]],
                buf_v[b % 2], sem_g.at[b % 2])

        pending_in = start_gather(0)
        pending_out = [None, None]
        for b in range(_BATCH):
            s = b % 2
            if b + 1 < _BATCH:
                if pending_out[1 - s] is not None:
                    pending_out[1 - s].wait()
                    pending_out[1 - s] = None
                nxt = start_gather(b + 1)
            pending_in.wait()
            if b + 1 < _BATCH:
                pending_in = nxt
            pending_out[s] = pltpu.async_copy(
                buf_v[s], out_hbm.at[b, pl.ds(pbase, _CHUNK)], sem_w.at[s])
        for w in pending_out:
            if w is not None:
                w.wait()

    return body(idx_flat, table)


def kernel(x, W):
    xt = x.T
    parts = []
    for k in range(_K):
        idx_k = lax.slice(xt, (0, k * _SP), (_BATCH, (k + 1) * _SP))
        g = _gather(idx_k.reshape(_BATCH * _SP), W)
        parts.append(g.transpose(1, 0, 2) + jnp.asarray(_PE[k * _SP:(k + 1) * _SP]))
    return jnp.concatenate(parts, axis=0)


# speedup vs baseline: 1.1045x; 1.1045x over previous
import numpy as np
import jax
import jax.numpy as jnp
from jax import lax
from jax.experimental import pallas as pl
from jax.experimental.pallas import tpu as pltpu
from jax.experimental.pallas import tpu_sc as plsc

_VOCAB = 100000
_D = 768
_SEQ = 2048
_BATCH = 4

_NC, _NS = 2, 16
_NW = _NC * _NS
_K = 2
_SP = _SEQ
_PPW = _SP // _NW
_CHUNK = _PPW


def _pe_table() -> np.ndarray:
    position = np.arange(0.0, _SEQ)[:, None]
    div_term = np.exp(np.arange(0.0, _D, 2) * -(np.log(10000.0) / _D))
    pe = np.zeros((_SEQ, _D), dtype=np.float32)
    pe[:, 0::2] = np.sin(position * div_term)
    pe[:, 1::2] = np.cos(position * div_term)
    return pe[:, None, :]


_PE = _pe_table()

_mesh = plsc.VectorSubcoreMesh(core_axis_name="c", subcore_axis_name="s")


def _gather(idx_flat, table):
    @pl.kernel(
        out_type=jax.ShapeDtypeStruct((_BATCH, _SP, _D), jnp.float32),
        mesh=_mesh,
        scratch_types=[
            pltpu.VMEM((_BATCH * _PPW,), jnp.int32),
            pltpu.VMEM((_CHUNK, _D), jnp.float32),
            pltpu.VMEM((_CHUNK, _D), jnp.float32),
            pltpu.SemaphoreType.DMA((_BATCH,)),
            pltpu.SemaphoreType.DMA((2,)),
            pltpu.SemaphoreType.DMA((2,)),
        ],
    )
    def body(idx_hbm, table_hbm, out_hbm, idx_v, buf_v0, buf_v1,
             sem_i, sem_g, sem_w):
        buf_v = [buf_v0, buf_v1]
        wid = lax.axis_index("s") * _NC + lax.axis_index("c")
        pbase = wid * _PPW
        idx_cp = [
            pltpu.async_copy(
                idx_hbm.at[pl.ds(b * _SP + pbase, _PPW)],
                idx_v.at[pl.ds(b * _PPW, _PPW)], sem_i.at[b])
            for b in range(_BATCH)
        ]

        def start_gather(b):
            idx_cp[b].wait()
            return pltpu.async_copy(
                table_hbm.at[idx_v.at[pl.ds(b * _PPW, _CHUNK)]],
                buf_v[b % 2], sem_g.at[b % 2])

        pending_in = start_gather(0)
        pending_out = [None, None]
        for b in range(_BATCH):
            s = b % 2
            if b + 1 < _BATCH:
                if pending_out[1 - s] is not None:
                    pending_out[1 - s].wait()
                    pending_out[1 - s] = None
                nxt = start_gather(b + 1)
            pending_in.wait()
            if b + 1 < _BATCH:
                pending_in = nxt
            pending_out[s] = pltpu.async_copy(
                buf_v[s], out_hbm.at[b, pl.ds(pbase, _CHUNK)], sem_w.at[s])
        for w in pending_out:
            if w is not None:
                w.wait()

    return body(idx_flat, table)


def kernel(x, W):
    idx_flat = x.T.reshape(_BATCH * _SEQ)
    g = _gather(idx_flat, W)
    h = _SEQ // _K
    parts = []
    for k in range(_K):
        gk = lax.slice(g, (0, k * h, 0), (_BATCH, (k + 1) * h, _D))
        parts.append(gk.transpose(1, 0, 2) + jnp.asarray(_PE[k * h:(k + 1) * h]))
    return jnp.concatenate(parts, axis=0)


# speedup vs baseline: 1.1277x; 1.0210x over previous
import numpy as np
import jax
import jax.numpy as jnp
from jax import lax
from jax.experimental import pallas as pl
from jax.experimental.pallas import tpu as pltpu
from jax.experimental.pallas import tpu_sc as plsc

_VOCAB = 100000
_D = 768
_SEQ = 2048
_BATCH = 4

_NC, _NS = 2, 16
_NW = _NC * _NS
_PPW = _SEQ // _NW
_R = 8
_NCH = _PPW // _R
_ROWS = _R * _BATCH
_LANES = 16


def _pe_table() -> np.ndarray:
    position = np.arange(0.0, _SEQ)[:, None]
    div_term = np.exp(np.arange(0.0, _D, 2) * -(np.log(10000.0) / _D))
    pe = np.zeros((_SEQ, _D), dtype=np.float32)
    pe[:, 0::2] = np.sin(position * div_term)
    pe[:, 1::2] = np.cos(position * div_term)
    return pe


_PE = _pe_table()

_mesh = plsc.VectorSubcoreMesh(core_axis_name="c", subcore_axis_name="s")


@pl.kernel(
    out_type=jax.ShapeDtypeStruct((_SEQ * _BATCH, _D), jnp.float32),
    mesh=_mesh,
    scratch_types=[
        pltpu.VMEM((_PPW * _BATCH,), jnp.int32),
        pltpu.VMEM((_PPW, _D), jnp.float32),
        pltpu.VMEM((_ROWS, _D), jnp.float32),
        pltpu.VMEM((_ROWS, _D), jnp.float32),
        pltpu.SemaphoreType.DMA,
        pltpu.SemaphoreType.DMA,
        pltpu.SemaphoreType.DMA((2,)),
        pltpu.SemaphoreType.DMA((2,)),
    ],
)
def _fused(idx_hbm, pe_hbm, table_hbm, out_hbm, idx_v, pe_v,
           buf_v0, buf_v1, sem_i, sem_p, sem_g, sem_w):
    buf_v = [buf_v0, buf_v1]
    wid = lax.axis_index("s") * _NC + lax.axis_index("c")
    pbase = wid * _PPW
    idx_cp = pltpu.async_copy(
        idx_hbm.at[pl.ds(pbase * _BATCH, _PPW * _BATCH)], idx_v, sem_i)
    pe_cp = pltpu.async_copy(pe_hbm.at[pl.ds(pbase, _PPW)], pe_v, sem_p)

    def start_gather(c):
        return pltpu.async_copy(
            table_hbm.at[idx_v.at[pl.ds(c * _ROWS, _ROWS)]],
            buf_v[c % 2], sem_g.at[c % 2])

    def add_pe(c, buf):
        def sbody(s_l, _):
            def dbody(d, _):
                pv = pe_v[c * _R + s_l, pl.ds(d, _LANES)]
                for b in range(_BATCH):
                    r = s_l * _BATCH + b
                    buf[r, pl.ds(d, _LANES)] = buf[r, pl.ds(d, _LANES)] + pv
                return 0
            return lax.fori_loop(0, _D // _LANES,
                                 lambda j, u: dbody(j * _LANES, u), 0)
        lax.fori_loop(0, _R, sbody, 0)

    idx_cp.wait()
    pending_in = start_gather(0)
    pending_out = [None, None]
    pe_cp.wait()
    for c in range(_NCH):
        s = c % 2
        if c + 1 < _NCH:
            if pending_out[1 - s] is not None:
                pending_out[1 - s].wait()
                pending_out[1 - s] = None
            nxt = start_gather(c + 1)
        pending_in.wait()
        if c + 1 < _NCH:
            pending_in = nxt
        add_pe(c, buf_v[s])
        pending_out[s] = pltpu.async_copy(
            buf_v[s],
            out_hbm.at[pl.ds((pbase + c * _R) * _BATCH, _ROWS)],
            sem_w.at[s])
    for w in pending_out:
        if w is not None:
            w.wait()


def kernel(x, W):
    idx_flat = x.reshape(_SEQ * _BATCH)
    pe = jnp.asarray(_PE)
    out = _fused(idx_flat, pe, W)
    return out.reshape(_SEQ, _BATCH, _D)


# speedup vs baseline: 1.2577x; 1.1153x over previous
import numpy as np
import jax
import jax.numpy as jnp
from jax import lax
from jax.experimental import pallas as pl
from jax.experimental.pallas import tpu as pltpu
from jax.experimental.pallas import tpu_sc as plsc

_VOCAB = 100000
_D = 768
_SEQ = 2048
_BATCH = 4

_NC, _NS = 2, 16
_NW = _NC * _NS
_PPW = _SEQ // _NW
_R = 8
_NCH = _PPW // _R
_ROWS = _R * _BATCH
_LANES = 16


def _pe_table() -> np.ndarray:
    position = np.arange(0.0, _SEQ)[:, None]
    div_term = np.exp(np.arange(0.0, _D, 2) * -(np.log(10000.0) / _D))
    pe = np.zeros((_SEQ, _D), dtype=np.float32)
    pe[:, 0::2] = np.sin(position * div_term)
    pe[:, 1::2] = np.cos(position * div_term)
    return pe


_PE = _pe_table()

_mesh = plsc.VectorSubcoreMesh(core_axis_name="c", subcore_axis_name="s")


@pl.kernel(
    out_type=jax.ShapeDtypeStruct((_SEQ * _BATCH, _D), jnp.float32),
    mesh=_mesh,
    scratch_types=[
        pltpu.VMEM((_PPW * _BATCH,), jnp.int32),
        pltpu.VMEM((_PPW, _D), jnp.float32),
        pltpu.VMEM((_ROWS, _D), jnp.float32),
        pltpu.VMEM((_ROWS, _D), jnp.float32),
        pltpu.SemaphoreType.DMA,
        pltpu.SemaphoreType.DMA,
        pltpu.SemaphoreType.DMA((2,)),
        pltpu.SemaphoreType.DMA((2,)),
    ],
)
def _fused(idx_hbm, pe_hbm, table_hbm, out_hbm, idx_v, pe_v,
           buf_v0, buf_v1, sem_i, sem_p, sem_g, sem_w):
    buf_v = [buf_v0, buf_v1]
    wid = lax.axis_index("s") * _NC + lax.axis_index("c")
    pbase = wid * _PPW
    idx_cp = pltpu.async_copy(
        idx_hbm.at[pl.ds(pbase * _BATCH, _PPW * _BATCH)], idx_v, sem_i)
    pe_cp = pltpu.async_copy(pe_hbm.at[pl.ds(pbase, _PPW)], pe_v, sem_p)

    def start_gather(c):
        return pltpu.async_copy(
            table_hbm.at[idx_v.at[pl.ds(c * _ROWS, _ROWS)]],
            buf_v[c % 2], sem_g.at[c % 2])

    def add_pe(c, buf):
        def sbody(s_l, _):
            @plsc.parallel_loop(0, _D, _LANES, unroll=4)
            def dloop(d):
                pv = pe_v[c * _R + s_l, pl.ds(d, _LANES)]
                for b in range(_BATCH):
                    r = s_l * _BATCH + b
                    buf[r, pl.ds(d, _LANES)] = buf[r, pl.ds(d, _LANES)] + pv
            return 0
        lax.fori_loop(0, _R, sbody, 0)

    idx_cp.wait()
    pending_in = start_gather(0)
    pending_out = [None, None]
    pe_cp.wait()
    for c in range(_NCH):
        s = c % 2
        if c + 1 < _NCH:
            if pending_out[1 - s] is not None:
                pending_out[1 - s].wait()
                pending_out[1 - s] = None
            nxt = start_gather(c + 1)
        pending_in.wait()
        if c + 1 < _NCH:
            pending_in = nxt
        add_pe(c, buf_v[s])
        pending_out[s] = pltpu.async_copy(
            buf_v[s],
            out_hbm.at[pl.ds((pbase + c * _R) * _BATCH, _ROWS)],
            sem_w.at[s])
    for w in pending_out:
        if w is not None:
            w.wait()


def kernel(x, W):
    idx_flat = x.reshape(_SEQ * _BATCH)
    pe = jnp.asarray(_PE)
    out = _fused(idx_flat, pe, W)
    return out.reshape(_SEQ, _BATCH, _D)


# speedup vs baseline: 1.2581x; 1.0003x over previous
import numpy as np
import jax
import jax.numpy as jnp
from jax import lax
from jax.experimental import pallas as pl
from jax.experimental.pallas import tpu as pltpu
from jax.experimental.pallas import tpu_sc as plsc

_VOCAB = 100000
_D = 768
_SEQ = 2048
_BATCH = 4

_NC, _NS = 2, 16
_NW = _NC * _NS
_PPW = _SEQ // _NW
_R = 8
_NCH = _PPW // _R
_ROWS = _R * _BATCH
_LANES = 16


def _pe_table() -> np.ndarray:
    position = np.arange(0.0, _SEQ)[:, None]
    div_term = np.exp(np.arange(0.0, _D, 2) * -(np.log(10000.0) / _D))
    pe = np.zeros((_SEQ, _D), dtype=np.float32)
    pe[:, 0::2] = np.sin(position * div_term)
    pe[:, 1::2] = np.cos(position * div_term)
    return pe


_PE = _pe_table()

_mesh = plsc.VectorSubcoreMesh(core_axis_name="c", subcore_axis_name="s")


@pl.kernel(
    out_type=jax.ShapeDtypeStruct((_SEQ * _BATCH, _D), jnp.float32),
    mesh=_mesh,
    scratch_types=[
        pltpu.VMEM((_PPW * _BATCH,), jnp.int32),
        pltpu.VMEM((_PPW, _D), jnp.float32),
        pltpu.VMEM((_ROWS, _D), jnp.float32),
        pltpu.VMEM((_ROWS, _D), jnp.float32),
        pltpu.SemaphoreType.DMA,
        pltpu.SemaphoreType.DMA,
        pltpu.SemaphoreType.DMA((2,)),
        pltpu.SemaphoreType.DMA((2,)),
    ],
)
def _fused(idx_hbm, pe_hbm, table_hbm, out_hbm, idx_v, pe_v,
           buf_v0, buf_v1, sem_i, sem_p, sem_g, sem_w):
    buf_v = [buf_v0, buf_v1]
    wid = lax.axis_index("s") * _NC + lax.axis_index("c")
    pbase = wid * _PPW
    idx_cp = pltpu.async_copy(
        idx_hbm.at[pl.ds(pbase * _BATCH, _PPW * _BATCH)], idx_v, sem_i)
    pe_cp = pltpu.async_copy(pe_hbm.at[pl.ds(pbase, _PPW)], pe_v, sem_p)

    def start_gather(c):
        return pltpu.async_copy(
            table_hbm.at[idx_v.at[pl.ds(c * _ROWS, _ROWS)]],
            buf_v[c % 2], sem_g.at[c % 2])

    def add_pe(c, buf):
        def sbody(s_l, _):
            @plsc.parallel_loop(0, _D, _LANES, unroll=8)
            def dloop(d):
                pv = pe_v[c * _R + s_l, pl.ds(d, _LANES)]
                for b in range(_BATCH):
                    r = s_l * _BATCH + b
                    buf[r, pl.ds(d, _LANES)] = buf[r, pl.ds(d, _LANES)] + pv
            return 0
        lax.fori_loop(0, _R, sbody, 0)

    idx_cp.wait()
    pending_in = start_gather(0)
    pending_out = [None, None]
    pe_cp.wait()
    for c in range(_NCH):
        s = c % 2
        if c + 1 < _NCH:
            if pending_out[1 - s] is not None:
                pending_out[1 - s].wait()
                pending_out[1 - s] = None
            nxt = start_gather(c + 1)
        pending_in.wait()
        if c + 1 < _NCH:
            pending_in = nxt
        add_pe(c, buf_v[s])
        pending_out[s] = pltpu.async_copy(
            buf_v[s],
            out_hbm.at[pl.ds((pbase + c * _R) * _BATCH, _ROWS)],
            sem_w.at[s])
    for w in pending_out:
        if w is not None:
            w.wait()


def kernel(x, W):
    idx_flat = x.reshape(_SEQ * _BATCH)
    pe = jnp.asarray(_PE)
    out = _fused(idx_flat, pe, W)
    return out.reshape(_SEQ, _BATCH, _D)
